# baseline (speedup 1.0000x reference)
import functools

import jax
import jax.numpy as jnp
from jax import lax
from jax.experimental import pallas as pl
from jax.experimental.pallas import tpu as pltpu
from jax.experimental.pallas import tpu_sc as plsc

N = 10000
NPAD = 10240
RPT = 640
NC = 2
NS = 16
NW = NC * NS
CH = 64
S_SIM = 160
S_COM = 80
BR = 1000
GRID = N // BR

_mesh = lambda: plsc.VectorSubcoreMesh(core_axis_name="c", subcore_axis_name="s")


def _pad_edges(edge_index, edge_weight, S):
    E = edge_weight.shape[0]
    Ep = NW * S * CH
    pad = Ep - E
    src = edge_index[0].astype(jnp.int32)
    dst = edge_index[1].astype(jnp.int32)
    pidx = (jnp.arange(pad, dtype=jnp.int32) * 997) % N
    src = jnp.concatenate([src, pidx]).reshape(NW * S, CH)
    dst = jnp.concatenate([dst, pidx]).reshape(NW * S, CH)
    w = jnp.concatenate([edge_weight, jnp.zeros((pad,), edge_weight.dtype)])
    wexp = jnp.broadcast_to(w[:, None], (Ep, 16))
    return src, dst, w.reshape(NW * S, CH), wexp


def _deg_kernel(dst_sim, w_sim, dst_dist, w_dist, dst_com, w_com, z1,
                out_sim, out_dist, out_com, idx_v, w_v, acc_sh, sem):
    del sem
    c = lax.axis_index("c")
    s = lax.axis_index("s")
    wid = s * NC + c
    for S, dst_h, w_h, out_h in ((S_SIM, dst_sim, w_sim, out_sim),
                                 (S_SIM, dst_dist, w_dist, out_dist),
                                 (S_COM, dst_com, w_com, out_com)):
        pltpu.sync_copy(z1, acc_sh.at[pl.ds(s * RPT, RPT)])
        plsc.subcore_barrier()
        pltpu.sync_copy(dst_h.at[pl.ds(wid * S, S)], idx_v.at[pl.ds(0, S)])
        pltpu.sync_copy(w_h.at[pl.ds(wid * S, S)], w_v.at[pl.ds(0, S)])

        def body(j, carry):
            pltpu.sync_copy(w_v.at[j], acc_sh.at[idx_v.at[j]], add=True)
            return carry

        lax.fori_loop(0, S, body, 0)
        plsc.subcore_barrier()
        pltpu.sync_copy(acc_sh.at[pl.ds(s * RPT, RPT)],
                        out_h.at[c, pl.ds(s * RPT, RPT)])
        plsc.subcore_barrier()


def _run_deg(dst_sim, w_sim, dst_dist, w_dist, dst_com, w_com):
    z1 = jnp.zeros((RPT,), jnp.float32)
    f = functools.partial(
        pl.kernel,
        mesh=_mesh(),
        out_type=[jax.ShapeDtypeStruct((NC, NPAD), jnp.float32)] * 3,
        scratch_types=[
            pltpu.VMEM((S_SIM, CH), jnp.int32),
            pltpu.VMEM((S_SIM, CH), jnp.float32),
            pltpu.VMEM_SHARED((NPAD,), jnp.float32),
            pltpu.SemaphoreType.DMA,
        ],
    )(_deg_kernel)
    return f(dst_sim, w_sim, dst_dist, w_dist, dst_com, w_com, z1)


def _agg_section(c, s, table_h, src_h, dst_h, w_h, out_h, z_h, S,
                 acc_sh, idx_s, idx_d, wv, rowsv, gsems, ssems):
    wid = s * NC + c
    pltpu.sync_copy(z_h, acc_sh.at[pl.ds(s * RPT, RPT)])
    plsc.subcore_barrier()

    def group(g, carry):
        base = wid * S + g * 8
        pltpu.sync_copy(src_h.at[pl.ds(base, 8)], idx_s)
        pltpu.sync_copy(dst_h.at[pl.ds(base, 8)], idx_d)

        def start_fetch(j, p):
            hg = pltpu.async_copy(table_h.at[idx_s.at[j]], rowsv[p], gsems[p])
            hw = pltpu.async_copy(w_h.at[pl.ds((base + j) * CH, CH)], wv[p],
                                  gsems[p])
            return (hg, hw)

        hg = start_fetch(0, 0)
        hs = [None, None]
        for j in range(8):
            p = j & 1
            if j + 1 < 8:
                if hs[1 - p] is not None:
                    hs[1 - p].wait()
                hg_next = start_fetch(j + 1, 1 - p)
            hg[0].wait()
            hg[1].wait()

            def rowscale(i, carry2, _p=p):
                wb = wv[_p][i, pl.ds(0, 16)]
                for f in range(128 // 16):
                    seg = rowsv[_p][i, pl.ds(f * 16, 16)]
                    rowsv[_p][i, pl.ds(f * 16, 16)] = seg * wb
                return carry2

            lax.fori_loop(0, CH, rowscale, 0, unroll=8)
            hs[p] = pltpu.async_copy(rowsv[p], acc_sh.at[idx_d.at[j]],
                                     ssems[p], add=True)
            if j + 1 < 8:
                hg = hg_next
        hs[0].wait()
        hs[1].wait()
        return carry

    lax.fori_loop(0, S // 8, group, 0)
    plsc.subcore_barrier()
    pltpu.sync_copy(acc_sh.at[pl.ds(s * RPT, RPT)],
                    out_h.at[c, pl.ds(s * RPT, RPT), :])
    plsc.subcore_barrier()


def _agg2_kernel(xws_sim, xws_dist, xws_pa,
                 src_sim, dst_sim, w_sim, src_dist, dst_dist, w_dist,
                 src_com, dst_com, w_com, z128,
                 out_sim, out_dist, out_pa,
                 idx_s, idx_d, w_a, w_b, rows_a, rows_b, acc_sh,
                 gsem_a, gsem_b, ssem_a, ssem_b):
    c = lax.axis_index("c")
    s = lax.axis_index("s")
    wv = (w_a, w_b)
    rowsv = (rows_a, rows_b)
    gsems = (gsem_a, gsem_b)
    ssems = (ssem_a, ssem_b)
    _agg_section(c, s, xws_sim, src_sim, dst_sim, w_sim, out_sim, z128,
                 S_SIM, acc_sh, idx_s, idx_d, wv, rowsv, gsems, ssems)
    _agg_section(c, s, xws_dist, src_dist, dst_dist, w_dist, out_dist, z128,
                 S_SIM, acc_sh, idx_s, idx_d, wv, rowsv, gsems, ssems)
    _agg_section(c, s, xws_pa, src_com, dst_com, w_com, out_pa, z128,
                 S_COM, acc_sh, idx_s, idx_d, wv, rowsv, gsems, ssems)


def _agg_scratch():
    return [
        pltpu.VMEM((8, CH), jnp.int32),
        pltpu.VMEM((8, CH), jnp.int32),
        pltpu.VMEM((CH, 16), jnp.float32),
        pltpu.VMEM((CH, 16), jnp.float32),
        pltpu.VMEM((CH, 128), jnp.float32),
        pltpu.VMEM((CH, 128), jnp.float32),
        pltpu.VMEM_SHARED((NPAD, 128), jnp.float32),
        pltpu.SemaphoreType.DMA,
        pltpu.SemaphoreType.DMA,
        pltpu.SemaphoreType.DMA,
        pltpu.SemaphoreType.DMA,
    ]


def _run_agg2(xws_sim, xws_dist, xws_pa, e_sim, e_dist, e_com):
    z128 = jnp.zeros((RPT, 128), jnp.float32)
    f = functools.partial(
        pl.kernel,
        mesh=_mesh(),
        out_type=[jax.ShapeDtypeStruct((NC, NPAD, 128), jnp.float32)] * 3,
        scratch_types=_agg_scratch(),
    )(_agg2_kernel)
    return f(xws_sim, xws_dist, xws_pa,
             e_sim[0], e_sim[1], e_sim[3],
             e_dist[0], e_dist[1], e_dist[3],
             e_com[0], e_com[1], e_com[3], z128)


def _agg3_kernel(xws2, src_sim, dst_sim, w_sim, src_dist, dst_dist, w_dist,
                 z128, out_sim, out_dist,
                 idx_s, idx_d, w_a, w_b, rows_a, rows_b, acc_sh,
                 gsem_a, gsem_b, ssem_a, ssem_b):
    c = lax.axis_index("c")
    s = lax.axis_index("s")
    wv = (w_a, w_b)
    rowsv = (rows_a, rows_b)
    gsems = (gsem_a, gsem_b)
    ssems = (ssem_a, ssem_b)
    _agg_section(c, s, xws2, src_sim, dst_sim, w_sim, out_sim, z128,
                 S_SIM, acc_sh, idx_s, idx_d, wv, rowsv, gsems, ssems)
    _agg_section(c, s, xws2, src_dist, dst_dist, w_dist, out_dist, z128,
                 S_SIM, acc_sh, idx_s, idx_d, wv, rowsv, gsems, ssems)


def _run_agg3(xws2, e_sim, e_dist):
    z128 = jnp.zeros((RPT, 128), jnp.float32)
    f = functools.partial(
        pl.kernel,
        mesh=_mesh(),
        out_type=[jax.ShapeDtypeStruct((NC, NPAD, 128), jnp.float32)] * 2,
        scratch_types=_agg_scratch(),
    )(_agg3_kernel)
    return f(xws2, e_sim[0], e_sim[1], e_sim[3],
             e_dist[0], e_dist[1], e_dist[3], z128)


def _dinv(dp):
    deg = dp[0] + dp[1] + 1.0
    return jnp.where(deg > 0, lax.rsqrt(deg), 0.0)


def _t1_body(dps_ref, dpd_ref, dpc_ref, x_ref, adt_ref, atac_ref,
             w1_ref, w2_ref, wp_ref, wa_ref,
             xws_s_ref, xws_d_ref, xws_pa_ref, dvs_ref, dvd_ref, dvc_ref):
    dvs = _dinv(dps_ref[...])
    dvd = _dinv(dpd_ref[...])
    dvc = _dinv(dpc_ref[...])
    dvs_ref[...] = dvs
    dvd_ref[...] = dvd
    dvc_ref[...] = dvc
    x = x_ref[...]
    xws_s_ref[...] = jnp.dot(x, w1_ref[...], preferred_element_type=jnp.float32) * dvs
    xws_d_ref[...] = jnp.dot(x, w2_ref[...], preferred_element_type=jnp.float32) * dvd
    p = jnp.dot(adt_ref[...], wp_ref[...], preferred_element_type=jnp.float32) * dvc
    a = jnp.dot(atac_ref[...], wa_ref[...], preferred_element_type=jnp.float32) * dvc
    xws_pa_ref[...] = jnp.concatenate(
        [p, a, jnp.zeros((p.shape[0], 64), jnp.float32)], axis=1)


def _full(shape):
    nd = len(shape)
    return pl.BlockSpec(shape, lambda i: (0,) * nd)


def _rows(shape):
    nd = len(shape)
    return pl.BlockSpec(shape, lambda i: (i,) + (0,) * (nd - 1))


def _run_t1(dgp_sim, dgp_dist, dgp_com, x_rna, x_adt, x_atac, W1, W2, Wp, Wa):
    dps = dgp_sim[:, :N].reshape(NC, N, 1)
    dpd = dgp_dist[:, :N].reshape(NC, N, 1)
    dpc = dgp_com[:, :N].reshape(NC, N, 1)
    out_shape = [jax.ShapeDtypeStruct((N, 128), jnp.float32),
                 jax.ShapeDtypeStruct((N, 128), jnp.float32),
                 jax.ShapeDtypeStruct((N, 128), jnp.float32),
                 jax.ShapeDtypeStruct((N, 1), jnp.float32),
                 jax.ShapeDtypeStruct((N, 1), jnp.float32),
                 jax.ShapeDtypeStruct((N, 1), jnp.float32)]
    deg_spec = pl.BlockSpec((NC, BR, 1), lambda i: (0, i, 0))
    in_specs = [
        deg_spec, deg_spec, deg_spec,
        _rows((BR, 128)), _rows((BR, 32)), _rows((BR, 64)),
        _full((128, 128)), _full((128, 128)), _full((32, 32)), _full((64, 32)),
    ]
    out_specs = [_rows((BR, 128)), _rows((BR, 128)), _rows((BR, 128)),
                 _rows((BR, 1)), _rows((BR, 1)), _rows((BR, 1))]
    return pl.pallas_call(_t1_body, grid=(GRID,), in_specs=in_specs,
                          out_specs=out_specs, out_shape=out_shape)(
        dps, dpd, dpc, x_rna, x_adt, x_atac, W1, W2, Wp, Wa)


def _t2_body(accs_ref, accd_ref, xws_s_ref, xws_d_ref, dvs_ref, dvd_ref,
             b1_ref, b2_ref, ws_ref, wd_ref, xws2_ref):
    dvs = dvs_ref[...]
    dvd = dvd_ref[...]
    xs = jnp.maximum(dvs * (accs_ref[0] + accs_ref[1] + xws_s_ref[...]) + b1_ref[...], 0.0)
    xd = jnp.maximum(dvd * (accd_ref[0] + accd_ref[1] + xws_d_ref[...]) + b2_ref[...], 0.0)
    s2 = jnp.dot(xs, ws_ref[...], preferred_element_type=jnp.float32) * dvs
    d2 = jnp.dot(xd, wd_ref[...], preferred_element_type=jnp.float32) * dvd
    xws2_ref[...] = jnp.concatenate(
        [s2, d2, jnp.zeros((s2.shape[0], 64), jnp.float32)], axis=1)


def _run_t2(acc_sim, acc_dist, xws_sim, xws_dist, dvs, dvd, b1, b2, Ws, Wd):
    out_shape = [jax.ShapeDtypeStruct((N, 128), jnp.float32)]
    acc_spec = pl.BlockSpec((NC, BR, 128), lambda i: (0, i, 0))
    in_specs = [
        acc_spec, acc_spec,
        _rows((BR, 128)), _rows((BR, 128)), _rows((BR, 1)), _rows((BR, 1)),
        _full((1, 128)), _full((1, 128)), _full((128, 32)), _full((128, 32)),
    ]
    out_specs = [_rows((BR, 128))]
    return pl.pallas_call(_t2_body, grid=(GRID,), in_specs=in_specs,
                          out_specs=out_specs, out_shape=out_shape)(
        acc_sim[:, :N], acc_dist[:, :N], xws_sim, xws_dist, dvs, dvd,
        b1.reshape(1, 128), b2.reshape(1, 128), Ws, Wd)[0]


def _t3_body(acc2s_ref, acc2d_ref, accpa_ref, xws2_ref, xws_pa_ref,
             dvs_ref, dvd_ref, dvc_ref,
             bs_ref, bd_ref, bp_ref, ba_ref, bf_ref,
             wf1_ref, wf2_ref, wf3_ref, wf4_ref,
             xsim_ref, xdist_ref, fused_ref, pro_ref, atac_ref):
    dvs = dvs_ref[...]
    dvd = dvd_ref[...]
    dvc = dvc_ref[...]
    a2s = acc2s_ref[0, :, 0:32] + acc2s_ref[1, :, 0:32]
    a2d = acc2d_ref[0, :, 32:64] + acc2d_ref[1, :, 32:64]
    ap = accpa_ref[0, :, 0:32] + accpa_ref[1, :, 0:32]
    aa = accpa_ref[0, :, 32:64] + accpa_ref[1, :, 32:64]
    x_sim = dvs * (a2s + xws2_ref[:, 0:32]) + bs_ref[...]
    x_dist = dvd * (a2d + xws2_ref[:, 32:64]) + bd_ref[...]
    pro = dvc * (ap + xws_pa_ref[:, 0:32]) + bp_ref[...]
    atac = dvc * (aa + xws_pa_ref[:, 32:64]) + ba_ref[...]
    xsim_ref[...] = x_sim
    xdist_ref[...] = x_dist
    pro_ref[...] = pro
    atac_ref[...] = atac
    fused_ref[...] = (
        jnp.dot(x_sim, wf1_ref[...], preferred_element_type=jnp.float32)
        + jnp.dot(x_dist, wf2_ref[...], preferred_element_type=jnp.float32)
        + jnp.dot(pro, wf3_ref[...], preferred_element_type=jnp.float32)
        + jnp.dot(atac, wf4_ref[...], preferred_element_type=jnp.float32)
        + bf_ref[...])


def _run_t3(acc2_sim, acc2_dist, acc_pa, xws2, xws_pa, dvs, dvd, dvc,
            bs, bd, bp, ba, bf, Wf):
    out_shape = [jax.ShapeDtypeStruct((N, 32), jnp.float32)] * 5
    acc_spec = pl.BlockSpec((NC, BR, 128), lambda i: (0, i, 0))
    in_specs = [acc_spec] * 3 + [_rows((BR, 128))] * 2 + [_rows((BR, 1))] * 3 \
        + [_full((1, 32))] * 5 + [_full((32, 32))] * 4
    out_specs = [_rows((BR, 32))] * 5
    return pl.pallas_call(_t3_body, grid=(GRID,), in_specs=in_specs,
                          out_specs=out_specs, out_shape=out_shape)(
        acc2_sim[:, :N], acc2_dist[:, :N], acc_pa[:, :N], xws2, xws_pa,
        dvs, dvd, dvc,
        bs.reshape(1, 32), bd.reshape(1, 32), bp.reshape(1, 32),
        ba.reshape(1, 32), bf.reshape(1, 32),
        Wf[0:32], Wf[32:64], Wf[64:96], Wf[96:128])


def kernel(x_RNA, x_ADT, x_ATAC, sim_edge_index, sim_edge_weight,
           dist_edge_index, dist_edge_weight, common_edge_index,
           common_edge_weight, W_rna1, b_rna1, W_rna2, b_rna2, W_sim, b_sim,
           W_dist, b_dist, W_pro, b_pro, W_atac, b_atac, W_fuse, b_fuse):
    e_sim = _pad_edges(sim_edge_index, sim_edge_weight, S_SIM)
    e_dist = _pad_edges(dist_edge_index, dist_edge_weight, S_SIM)
    e_com = _pad_edges(common_edge_index, common_edge_weight, S_COM)

    dgp_sim, dgp_dist, dgp_com = _run_deg(
        e_sim[1], e_sim[2], e_dist[1], e_dist[2], e_com[1], e_com[2])

    xws_sim, xws_dist, xws_pa, dvs, dvd, dvc = _run_t1(
        dgp_sim, dgp_dist, dgp_com, x_RNA, x_ADT, x_ATAC,
        W_rna1, W_rna2, W_pro, W_atac)

    acc_sim, acc_dist, acc_pa = _run_agg2(
        xws_sim, xws_dist, xws_pa, e_sim, e_dist, e_com)

    xws2 = _run_t2(acc_sim, acc_dist, xws_sim, xws_dist, dvs, dvd,
                   b_rna1, b_rna2, W_sim, W_dist)

    acc2_sim, acc2_dist = _run_agg3(xws2, e_sim, e_dist)

    x_sim, x_dist, fused, pro, atac = _run_t3(
        acc2_sim, acc2_dist, acc_pa, xws2, xws_pa, dvs, dvd, dvc,
        b_sim, b_dist, b_pro, b_atac, b_fuse, W_fuse)
    return (x_sim, x_dist, fused, pro, atac)


# speedup vs baseline: 1.0747x; 1.0747x over previous
import functools

import jax
import jax.numpy as jnp
from jax import lax
from jax.experimental import pallas as pl
from jax.experimental.pallas import tpu as pltpu
from jax.experimental.pallas import tpu_sc as plsc

N = 10000
NPAD = 10112
RPT = 632
NC = 2
NS = 16
NW = NC * NS
CH = 64
S_SIM = 160
S_COM = 80
G = 8
NB = 3
CHD = 128
NPADD = 10240
RPTD = 640
SD_SIM = 80
SD_COM = 40
BR = 1000
GRID = N // BR

_mesh = lambda: plsc.VectorSubcoreMesh(core_axis_name="c", subcore_axis_name="s")


def _pad_edges(edge_index, edge_weight, S):
    E = edge_weight.shape[0]
    Ep = NW * S * CH
    pad = Ep - E
    src = edge_index[0].astype(jnp.int32)
    dst = edge_index[1].astype(jnp.int32)
    pidx = (jnp.arange(pad, dtype=jnp.int32) * 997) % N
    src = jnp.concatenate([src, pidx])
    dst = jnp.concatenate([dst, pidx])
    w = jnp.concatenate([edge_weight, jnp.zeros((pad,), edge_weight.dtype)])
    wexp = jnp.broadcast_to(w[:, None], (Ep, 16))
    return (src.reshape(NW * S, CH), dst.reshape(NW * S, CH), wexp,
            dst.reshape(Ep // CHD, CHD), w.reshape(Ep // CHD, CHD))


def _deg_kernel(dst_sim, w_sim, dst_dist, w_dist, dst_com, w_com, z1,
                out_sim, out_dist, out_com, idx_v, w_v, acc_sh, sem):
    del sem
    c = lax.axis_index("c")
    s = lax.axis_index("s")
    wid = s * NC + c
    for S, dst_h, w_h, out_h in ((SD_SIM, dst_sim, w_sim, out_sim),
                                 (SD_SIM, dst_dist, w_dist, out_dist),
                                 (SD_COM, dst_com, w_com, out_com)):
        pltpu.sync_copy(z1, acc_sh.at[pl.ds(s * RPTD, RPTD)])
        plsc.subcore_barrier()
        pltpu.sync_copy(dst_h.at[pl.ds(wid * S, S)], idx_v.at[pl.ds(0, S)])
        pltpu.sync_copy(w_h.at[pl.ds(wid * S, S)], w_v.at[pl.ds(0, S)])

        def body(j, carry):
            pltpu.sync_copy(w_v.at[j], acc_sh.at[idx_v.at[j]], add=True)
            return carry

        lax.fori_loop(0, S, body, 0)
        plsc.subcore_barrier()
        pltpu.sync_copy(acc_sh.at[pl.ds(s * RPTD, RPTD)],
                        out_h.at[c, pl.ds(s * RPTD, RPTD)])
        plsc.subcore_barrier()


def _run_deg(dst_sim, w_sim, dst_dist, w_dist, dst_com, w_com):
    z1 = jnp.zeros((RPTD,), jnp.float32)
    f = functools.partial(
        pl.kernel,
        mesh=_mesh(),
        out_type=[jax.ShapeDtypeStruct((NC, NPADD), jnp.float32)] * 3,
        scratch_types=[
            pltpu.VMEM((SD_SIM, CHD), jnp.int32),
            pltpu.VMEM((SD_SIM, CHD), jnp.float32),
            pltpu.VMEM_SHARED((NPADD,), jnp.float32),
            pltpu.SemaphoreType.DMA,
        ],
    )(_deg_kernel)
    return f(dst_sim, w_sim, dst_dist, w_dist, dst_com, w_com, z1)


def _agg_section(c, s, table_h, src_h, dst_h, w_h, out_h, S,
                 acc_sh, idx_s, idx_d, wv, rowsv, gsems, wsems, ssems):
    wid = s * NC + c

    def zrow(i, carry):
        for f in range(128 // 16):
            rowsv[0][i, pl.ds(f * 16, 16)] = jnp.zeros((16,), jnp.float32)
        return carry

    lax.fori_loop(0, CH, zrow, 0)
    for k in range(RPT // CH):
        pltpu.sync_copy(rowsv[0], acc_sh.at[pl.ds(s * RPT + k * CH, CH)])
    rem = RPT % CH
    if rem:
        pltpu.sync_copy(rowsv[0].at[pl.ds(0, rem)],
                        acc_sh.at[pl.ds(s * RPT + (RPT // CH) * CH, rem)])
    plsc.subcore_barrier()

    def group(g, carry):
        base = wid * S + g * G
        pltpu.sync_copy(src_h.at[pl.ds(base, G)], idx_s)
        pltpu.sync_copy(dst_h.at[pl.ds(base, G)], idx_d)

        def fetch_rows(j, p):
            return pltpu.async_copy(table_h.at[idx_s.at[j]], rowsv[p],
                                    gsems[p])

        def fetch_w(j, p):
            return pltpu.async_copy(w_h.at[pl.ds((base + j) * CH, CH)], wv[p],
                                    wsems[p])

        depth = NB - 1
        hg = [None] * NB
        hs = [None] * NB
        hw = [None] * 2
        for k in range(depth):
            hg[k] = fetch_rows(k, k)
        hw[0] = fetch_w(0, 0)
        for j in range(G):
            p = j % NB
            wp = j % 2
            nj = j + depth
            if nj < G:
                q = nj % NB
                if hs[q] is not None:
                    hs[q].wait()
                    hs[q] = None
                hg[q] = fetch_rows(nj, q)
            if j + 1 < G:
                hw[1 - wp] = fetch_w(j + 1, 1 - wp)
            hg[p].wait()
            hw[wp].wait()

            def rowscale(i, carry2, _p=p, _wp=wp):
                wb = wv[_wp][i, pl.ds(0, 16)]
                for f in range(128 // 16):
                    seg = rowsv[_p][i, pl.ds(f * 16, 16)]
                    rowsv[_p][i, pl.ds(f * 16, 16)] = seg * wb
                return carry2

            lax.fori_loop(0, CH, rowscale, 0, unroll=4)
            hs[p] = pltpu.async_copy(rowsv[p], acc_sh.at[idx_d.at[j]],
                                     ssems[p], add=True)
        for q in range(NB):
            if hs[q] is not None:
                hs[q].wait()
        return carry

    lax.fori_loop(0, S // G, group, 0)
    plsc.subcore_barrier()
    pltpu.sync_copy(acc_sh.at[pl.ds(s * RPT, RPT)],
                    out_h.at[c, pl.ds(s * RPT, RPT), :])
    plsc.subcore_barrier()


def _agg2_kernel(xws_sim, xws_dist, xws_pa,
                 src_sim, dst_sim, w_sim, src_dist, dst_dist, w_dist,
                 src_com, dst_com, w_com,
                 out_sim, out_dist, out_pa,
                 idx_s, idx_d, w_a, w_b, rows_a, rows_b, rows_c, acc_sh,
                 gsem_a, gsem_b, gsem_c, wsem_a, wsem_b,
                 ssem_a, ssem_b, ssem_c):
    c = lax.axis_index("c")
    s = lax.axis_index("s")
    wv = (w_a, w_b)
    rowsv = (rows_a, rows_b, rows_c)
    gsems = (gsem_a, gsem_b, gsem_c)
    wsems = (wsem_a, wsem_b)
    ssems = (ssem_a, ssem_b, ssem_c)
    _agg_section(c, s, xws_sim, src_sim, dst_sim, w_sim, out_sim,
                 S_SIM, acc_sh, idx_s, idx_d, wv, rowsv, gsems, wsems, ssems)
    _agg_section(c, s, xws_dist, src_dist, dst_dist, w_dist, out_dist,
                 S_SIM, acc_sh, idx_s, idx_d, wv, rowsv, gsems, wsems, ssems)
    _agg_section(c, s, xws_pa, src_com, dst_com, w_com, out_pa,
                 S_COM, acc_sh, idx_s, idx_d, wv, rowsv, gsems, wsems, ssems)


def _agg_scratch():
    return ([pltpu.VMEM((G, CH), jnp.int32)] * 2
            + [pltpu.VMEM((CH, 16), jnp.float32)] * 2
            + [pltpu.VMEM((CH, 128), jnp.float32)] * NB
            + [pltpu.VMEM_SHARED((NPAD, 128), jnp.float32)]
            + [pltpu.SemaphoreType.DMA] * (NB + 2 + NB))


def _run_agg2(xws_sim, xws_dist, xws_pa, e_sim, e_dist, e_com):
    f = functools.partial(
        pl.kernel,
        mesh=_mesh(),
        out_type=[jax.ShapeDtypeStruct((NC, NPAD, 128), jnp.float32)] * 3,
        scratch_types=_agg_scratch(),
    )(_agg2_kernel)
    return f(xws_sim, xws_dist, xws_pa,
             e_sim[0], e_sim[1], e_sim[2],
             e_dist[0], e_dist[1], e_dist[2],
             e_com[0], e_com[1], e_com[2])


def _agg3_kernel(xws2, src_sim, dst_sim, w_sim, src_dist, dst_dist, w_dist,
                 out_sim, out_dist,
                 idx_s, idx_d, w_a, w_b, rows_a, rows_b, rows_c, acc_sh,
                 gsem_a, gsem_b, gsem_c, wsem_a, wsem_b,
                 ssem_a, ssem_b, ssem_c):
    c = lax.axis_index("c")
    s = lax.axis_index("s")
    wv = (w_a, w_b)
    rowsv = (rows_a, rows_b, rows_c)
    gsems = (gsem_a, gsem_b, gsem_c)
    wsems = (wsem_a, wsem_b)
    ssems = (ssem_a, ssem_b, ssem_c)
    _agg_section(c, s, xws2, src_sim, dst_sim, w_sim, out_sim,
                 S_SIM, acc_sh, idx_s, idx_d, wv, rowsv, gsems, wsems, ssems)
    _agg_section(c, s, xws2, src_dist, dst_dist, w_dist, out_dist,
                 S_SIM, acc_sh, idx_s, idx_d, wv, rowsv, gsems, wsems, ssems)


def _run_agg3(xws2, e_sim, e_dist):
    f = functools.partial(
        pl.kernel,
        mesh=_mesh(),
        out_type=[jax.ShapeDtypeStruct((NC, NPAD, 128), jnp.float32)] * 2,
        scratch_types=_agg_scratch(),
    )(_agg3_kernel)
    return f(xws2, e_sim[0], e_sim[1], e_sim[2],
             e_dist[0], e_dist[1], e_dist[2])


def _dinv(dp):
    deg = dp[0] + dp[1] + 1.0
    return jnp.where(deg > 0, lax.rsqrt(deg), 0.0)


def _t1_body(dps_ref, dpd_ref, dpc_ref, x_ref, adt_ref, atac_ref,
             w1_ref, w2_ref, wp_ref, wa_ref,
             xws_s_ref, xws_d_ref, xws_pa_ref, dvs_ref, dvd_ref, dvc_ref):
    dvs = _dinv(dps_ref[...])
    dvd = _dinv(dpd_ref[...])
    dvc = _dinv(dpc_ref[...])
    dvs_ref[...] = dvs
    dvd_ref[...] = dvd
    dvc_ref[...] = dvc
    x = x_ref[...]
    xws_s_ref[...] = jnp.dot(x, w1_ref[...], preferred_element_type=jnp.float32) * dvs
    xws_d_ref[...] = jnp.dot(x, w2_ref[...], preferred_element_type=jnp.float32) * dvd
    p = jnp.dot(adt_ref[...], wp_ref[...], preferred_element_type=jnp.float32) * dvc
    a = jnp.dot(atac_ref[...], wa_ref[...], preferred_element_type=jnp.float32) * dvc
    xws_pa_ref[...] = jnp.concatenate(
        [p, a, jnp.zeros((p.shape[0], 64), jnp.float32)], axis=1)


def _full(shape):
    nd = len(shape)
    return pl.BlockSpec(shape, lambda i: (0,) * nd)


def _rows(shape):
    nd = len(shape)
    return pl.BlockSpec(shape, lambda i: (i,) + (0,) * (nd - 1))


def _run_t1(dgp_sim, dgp_dist, dgp_com, x_rna, x_adt, x_atac, W1, W2, Wp, Wa):
    dps = dgp_sim[:, :N].reshape(NC, N, 1)
    dpd = dgp_dist[:, :N].reshape(NC, N, 1)
    dpc = dgp_com[:, :N].reshape(NC, N, 1)
    out_shape = [jax.ShapeDtypeStruct((N, 128), jnp.float32),
                 jax.ShapeDtypeStruct((N, 128), jnp.float32),
                 jax.ShapeDtypeStruct((N, 128), jnp.float32),
                 jax.ShapeDtypeStruct((N, 1), jnp.float32),
                 jax.ShapeDtypeStruct((N, 1), jnp.float32),
                 jax.ShapeDtypeStruct((N, 1), jnp.float32)]
    deg_spec = pl.BlockSpec((NC, BR, 1), lambda i: (0, i, 0))
    in_specs = [
        deg_spec, deg_spec, deg_spec,
        _rows((BR, 128)), _rows((BR, 32)), _rows((BR, 64)),
        _full((128, 128)), _full((128, 128)), _full((32, 32)), _full((64, 32)),
    ]
    out_specs = [_rows((BR, 128)), _rows((BR, 128)), _rows((BR, 128)),
                 _rows((BR, 1)), _rows((BR, 1)), _rows((BR, 1))]
    return pl.pallas_call(_t1_body, grid=(GRID,), in_specs=in_specs,
                          out_specs=out_specs, out_shape=out_shape)(
        dps, dpd, dpc, x_rna, x_adt, x_atac, W1, W2, Wp, Wa)


def _t2_body(accs_ref, accd_ref, xws_s_ref, xws_d_ref, dvs_ref, dvd_ref,
             b1_ref, b2_ref, ws_ref, wd_ref, xws2_ref):
    dvs = dvs_ref[...]
    dvd = dvd_ref[...]
    xs = jnp.maximum(dvs * (accs_ref[0] + accs_ref[1] + xws_s_ref[...]) + b1_ref[...], 0.0)
    xd = jnp.maximum(dvd * (accd_ref[0] + accd_ref[1] + xws_d_ref[...]) + b2_ref[...], 0.0)
    s2 = jnp.dot(xs, ws_ref[...], preferred_element_type=jnp.float32) * dvs
    d2 = jnp.dot(xd, wd_ref[...], preferred_element_type=jnp.float32) * dvd
    xws2_ref[...] = jnp.concatenate(
        [s2, d2, jnp.zeros((s2.shape[0], 64), jnp.float32)], axis=1)


def _run_t2(acc_sim, acc_dist, xws_sim, xws_dist, dvs, dvd, b1, b2, Ws, Wd):
    out_shape = [jax.ShapeDtypeStruct((N, 128), jnp.float32)]
    acc_spec = pl.BlockSpec((NC, BR, 128), lambda i: (0, i, 0))
    in_specs = [
        acc_spec, acc_spec,
        _rows((BR, 128)), _rows((BR, 128)), _rows((BR, 1)), _rows((BR, 1)),
        _full((1, 128)), _full((1, 128)), _full((128, 32)), _full((128, 32)),
    ]
    out_specs = [_rows((BR, 128))]
    return pl.pallas_call(_t2_body, grid=(GRID,), in_specs=in_specs,
                          out_specs=out_specs, out_shape=out_shape)(
        acc_sim[:, :N], acc_dist[:, :N], xws_sim, xws_dist, dvs, dvd,
        b1.reshape(1, 128), b2.reshape(1, 128), Ws, Wd)[0]


def _t3_body(acc2s_ref, acc2d_ref, accpa_ref, xws2_ref, xws_pa_ref,
             dvs_ref, dvd_ref, dvc_ref,
             bs_ref, bd_ref, bp_ref, ba_ref, bf_ref,
             wf1_ref, wf2_ref, wf3_ref, wf4_ref,
             xsim_ref, xdist_ref, fused_ref, pro_ref, atac_ref):
    dvs = dvs_ref[...]
    dvd = dvd_ref[...]
    dvc = dvc_ref[...]
    a2s = acc2s_ref[0, :, 0:32] + acc2s_ref[1, :, 0:32]
    a2d = acc2d_ref[0, :, 32:64] + acc2d_ref[1, :, 32:64]
    ap = accpa_ref[0, :, 0:32] + accpa_ref[1, :, 0:32]
    aa = accpa_ref[0, :, 32:64] + accpa_ref[1, :, 32:64]
    x_sim = dvs * (a2s + xws2_ref[:, 0:32]) + bs_ref[...]
    x_dist = dvd * (a2d + xws2_ref[:, 32:64]) + bd_ref[...]
    pro = dvc * (ap + xws_pa_ref[:, 0:32]) + bp_ref[...]
    atac = dvc * (aa + xws_pa_ref[:, 32:64]) + ba_ref[...]
    xsim_ref[...] = x_sim
    xdist_ref[...] = x_dist
    pro_ref[...] = pro
    atac_ref[...] = atac
    fused_ref[...] = (
        jnp.dot(x_sim, wf1_ref[...], preferred_element_type=jnp.float32)
        + jnp.dot(x_dist, wf2_ref[...], preferred_element_type=jnp.float32)
        + jnp.dot(pro, wf3_ref[...], preferred_element_type=jnp.float32)
        + jnp.dot(atac, wf4_ref[...], preferred_element_type=jnp.float32)
        + bf_ref[...])


def _run_t3(acc2_sim, acc2_dist, acc_pa, xws2, xws_pa, dvs, dvd, dvc,
            bs, bd, bp, ba, bf, Wf):
    out_shape = [jax.ShapeDtypeStruct((N, 32), jnp.float32)] * 5
    acc_spec = pl.BlockSpec((NC, BR, 128), lambda i: (0, i, 0))
    in_specs = [acc_spec] * 3 + [_rows((BR, 128))] * 2 + [_rows((BR, 1))] * 3 \
        + [_full((1, 32))] * 5 + [_full((32, 32))] * 4
    out_specs = [_rows((BR, 32))] * 5
    return pl.pallas_call(_t3_body, grid=(GRID,), in_specs=in_specs,
                          out_specs=out_specs, out_shape=out_shape)(
        acc2_sim[:, :N], acc2_dist[:, :N], acc_pa[:, :N], xws2, xws_pa,
        dvs, dvd, dvc,
        bs.reshape(1, 32), bd.reshape(1, 32), bp.reshape(1, 32),
        ba.reshape(1, 32), bf.reshape(1, 32),
        Wf[0:32], Wf[32:64], Wf[64:96], Wf[96:128])


def kernel(x_RNA, x_ADT, x_ATAC, sim_edge_index, sim_edge_weight,
           dist_edge_index, dist_edge_weight, common_edge_index,
           common_edge_weight, W_rna1, b_rna1, W_rna2, b_rna2, W_sim, b_sim,
           W_dist, b_dist, W_pro, b_pro, W_atac, b_atac, W_fuse, b_fuse):
    e_sim = _pad_edges(sim_edge_index, sim_edge_weight, S_SIM)
    e_dist = _pad_edges(dist_edge_index, dist_edge_weight, S_SIM)
    e_com = _pad_edges(common_edge_index, common_edge_weight, S_COM)

    dgp_sim, dgp_dist, dgp_com = _run_deg(
        e_sim[3], e_sim[4], e_dist[3], e_dist[4], e_com[3], e_com[4])

    xws_sim, xws_dist, xws_pa, dvs, dvd, dvc = _run_t1(
        dgp_sim, dgp_dist, dgp_com, x_RNA, x_ADT, x_ATAC,
        W_rna1, W_rna2, W_pro, W_atac)

    acc_sim, acc_dist, acc_pa = _run_agg2(
        xws_sim, xws_dist, xws_pa, e_sim, e_dist, e_com)

    xws2 = _run_t2(acc_sim, acc_dist, xws_sim, xws_dist, dvs, dvd,
                   b_rna1, b_rna2, W_sim, W_dist)

    acc2_sim, acc2_dist = _run_agg3(xws2, e_sim, e_dist)

    x_sim, x_dist, fused, pro, atac = _run_t3(
        acc2_sim, acc2_dist, acc_pa, xws2, xws_pa, dvs, dvd, dvc,
        b_sim, b_dist, b_pro, b_atac, b_fuse, W_fuse)
    return (x_sim, x_dist, fused, pro, atac)


# speedup vs baseline: 1.0757x; 1.0009x over previous
import functools

import jax
import jax.numpy as jnp
from jax import lax
from jax.experimental import pallas as pl
from jax.experimental.pallas import tpu as pltpu
from jax.experimental.pallas import tpu_sc as plsc

N = 10000
NPAD = 10112
RPT = 632
NC = 2
NS = 16
NW = NC * NS
CH = 64
S_SIM = 160
S_COM = 80
G = 8
NB = 3
CHD = 128
NPADD = 10240
RPTD = 640
SD_SIM = 80
SD_COM = 40
BR = 1000
GRID = N // BR

_mesh = lambda: plsc.VectorSubcoreMesh(core_axis_name="c", subcore_axis_name="s")


def _pad_edges(edge_index, edge_weight, S):
    E = edge_weight.shape[0]
    Ep = NW * S * CH
    pad = Ep - E
    src = edge_index[0].astype(jnp.int32)
    dst = edge_index[1].astype(jnp.int32)
    pidx = (jnp.arange(pad, dtype=jnp.int32) * 997) % N
    src = jnp.concatenate([src, pidx])
    dst = jnp.concatenate([dst, pidx])
    w = jnp.concatenate([edge_weight, jnp.zeros((pad,), edge_weight.dtype)])
    wexp = jnp.broadcast_to(w[:, None], (Ep, 16))
    return (src.reshape(NW * S, CH), dst.reshape(NW * S, CH), wexp,
            dst.reshape(Ep // CHD, CHD), w.reshape(Ep // CHD, CHD))


def _deg_kernel(dst_sim, w_sim, dst_dist, w_dist, dst_com, w_com, z1,
                out_sim, out_dist, out_com, idx_v, w_v, acc_sh, sem):
    del sem
    c = lax.axis_index("c")
    s = lax.axis_index("s")
    wid = s * NC + c
    for S, dst_h, w_h, out_h in ((SD_SIM, dst_sim, w_sim, out_sim),
                                 (SD_SIM, dst_dist, w_dist, out_dist),
                                 (SD_COM, dst_com, w_com, out_com)):
        pltpu.sync_copy(z1, acc_sh.at[pl.ds(s * RPTD, RPTD)])
        plsc.subcore_barrier()
        pltpu.sync_copy(dst_h.at[pl.ds(wid * S, S)], idx_v.at[pl.ds(0, S)])
        pltpu.sync_copy(w_h.at[pl.ds(wid * S, S)], w_v.at[pl.ds(0, S)])

        def body(j, carry):
            pltpu.sync_copy(w_v.at[j], acc_sh.at[idx_v.at[j]], add=True)
            return carry

        lax.fori_loop(0, S, body, 0)
        plsc.subcore_barrier()
        pltpu.sync_copy(acc_sh.at[pl.ds(s * RPTD, RPTD)],
                        out_h.at[c, pl.ds(s * RPTD, RPTD)])
        plsc.subcore_barrier()


def _run_deg(dst_sim, w_sim, dst_dist, w_dist, dst_com, w_com):
    z1 = jnp.zeros((RPTD,), jnp.float32)
    f = functools.partial(
        pl.kernel,
        mesh=_mesh(),
        out_type=[jax.ShapeDtypeStruct((NC, NPADD), jnp.float32)] * 3,
        scratch_types=[
            pltpu.VMEM((SD_SIM, CHD), jnp.int32),
            pltpu.VMEM((SD_SIM, CHD), jnp.float32),
            pltpu.VMEM_SHARED((NPADD,), jnp.float32),
            pltpu.SemaphoreType.DMA,
        ],
    )(_deg_kernel)
    return f(dst_sim, w_sim, dst_dist, w_dist, dst_com, w_com, z1)


def _agg_section(c, s, table_h, src_h, dst_h, w_h, out_h, S,
                 acc_sh, idx_s, idx_d, wv, rowsv, gsems, wsems, ssems):
    wid = s * NC + c

    def zrow(i, carry):
        for f in range(128 // 16):
            rowsv[0][i, pl.ds(f * 16, 16)] = jnp.zeros((16,), jnp.float32)
        return carry

    lax.fori_loop(0, CH, zrow, 0)
    hz = [pltpu.async_copy(rowsv[0], acc_sh.at[pl.ds(s * RPT + k * CH, CH)],
                           gsems[0])
          for k in range(RPT // CH)]
    rem = RPT % CH
    if rem:
        hz.append(pltpu.async_copy(
            rowsv[0].at[pl.ds(0, rem)],
            acc_sh.at[pl.ds(s * RPT + (RPT // CH) * CH, rem)], gsems[0]))
    for h in hz:
        h.wait()
    plsc.subcore_barrier()

    def group(g, carry):
        base = wid * S + g * G
        pltpu.sync_copy(src_h.at[pl.ds(base, G)], idx_s)
        pltpu.sync_copy(dst_h.at[pl.ds(base, G)], idx_d)

        def fetch_rows(j, p):
            return pltpu.async_copy(table_h.at[idx_s.at[j]], rowsv[p],
                                    gsems[p])

        def fetch_w(j, p):
            return pltpu.async_copy(w_h.at[pl.ds((base + j) * CH, CH)], wv[p],
                                    wsems[p])

        depth = NB - 1
        hg = [None] * NB
        hs = [None] * NB
        hw = [None] * 2
        for k in range(depth):
            hg[k] = fetch_rows(k, k)
        hw[0] = fetch_w(0, 0)
        for j in range(G):
            p = j % NB
            wp = j % 2
            nj = j + depth
            if nj < G:
                q = nj % NB
                if hs[q] is not None:
                    hs[q].wait()
                    hs[q] = None
                hg[q] = fetch_rows(nj, q)
            if j + 1 < G:
                hw[1 - wp] = fetch_w(j + 1, 1 - wp)
            hg[p].wait()
            hw[wp].wait()

            def rowscale(i, carry2, _p=p, _wp=wp):
                wb = wv[_wp][i, pl.ds(0, 16)]
                for f in range(128 // 16):
                    seg = rowsv[_p][i, pl.ds(f * 16, 16)]
                    rowsv[_p][i, pl.ds(f * 16, 16)] = seg * wb
                return carry2

            lax.fori_loop(0, CH, rowscale, 0, unroll=4)
            hs[p] = pltpu.async_copy(rowsv[p], acc_sh.at[idx_d.at[j]],
                                     ssems[p], add=True)
        for q in range(NB):
            if hs[q] is not None:
                hs[q].wait()
        return carry

    lax.fori_loop(0, S // G, group, 0)
    plsc.subcore_barrier()
    pltpu.sync_copy(acc_sh.at[pl.ds(s * RPT, RPT)],
                    out_h.at[c, pl.ds(s * RPT, RPT), :])
    plsc.subcore_barrier()


def _agg2_kernel(xws_sim, xws_dist, xws_pa,
                 src_sim, dst_sim, w_sim, src_dist, dst_dist, w_dist,
                 src_com, dst_com, w_com,
                 out_sim, out_dist, out_pa,
                 idx_s, idx_d, w_a, w_b, rows_a, rows_b, rows_c, acc_sh,
                 gsem_a, gsem_b, gsem_c, wsem_a, wsem_b,
                 ssem_a, ssem_b, ssem_c):
    c = lax.axis_index("c")
    s = lax.axis_index("s")
    wv = (w_a, w_b)
    rowsv = (rows_a, rows_b, rows_c)
    gsems = (gsem_a, gsem_b, gsem_c)
    wsems = (wsem_a, wsem_b)
    ssems = (ssem_a, ssem_b, ssem_c)
    _agg_section(c, s, xws_sim, src_sim, dst_sim, w_sim, out_sim,
                 S_SIM, acc_sh, idx_s, idx_d, wv, rowsv, gsems, wsems, ssems)
    _agg_section(c, s, xws_dist, src_dist, dst_dist, w_dist, out_dist,
                 S_SIM, acc_sh, idx_s, idx_d, wv, rowsv, gsems, wsems, ssems)
    _agg_section(c, s, xws_pa, src_com, dst_com, w_com, out_pa,
                 S_COM, acc_sh, idx_s, idx_d, wv, rowsv, gsems, wsems, ssems)


def _agg_scratch():
    return ([pltpu.VMEM((G, CH), jnp.int32)] * 2
            + [pltpu.VMEM((CH, 16), jnp.float32)] * 2
            + [pltpu.VMEM((CH, 128), jnp.float32)] * NB
            + [pltpu.VMEM_SHARED((NPAD, 128), jnp.float32)]
            + [pltpu.SemaphoreType.DMA] * (NB + 2 + NB))


def _run_agg2(xws_sim, xws_dist, xws_pa, e_sim, e_dist, e_com):
    f = functools.partial(
        pl.kernel,
        mesh=_mesh(),
        out_type=[jax.ShapeDtypeStruct((NC, NPAD, 128), jnp.float32)] * 3,
        scratch_types=_agg_scratch(),
    )(_agg2_kernel)
    return f(xws_sim, xws_dist, xws_pa,
             e_sim[0], e_sim[1], e_sim[2],
             e_dist[0], e_dist[1], e_dist[2],
             e_com[0], e_com[1], e_com[2])


def _agg3_kernel(xws2, src_sim, dst_sim, w_sim, src_dist, dst_dist, w_dist,
                 out_sim, out_dist,
                 idx_s, idx_d, w_a, w_b, rows_a, rows_b, rows_c, acc_sh,
                 gsem_a, gsem_b, gsem_c, wsem_a, wsem_b,
                 ssem_a, ssem_b, ssem_c):
    c = lax.axis_index("c")
    s = lax.axis_index("s")
    wv = (w_a, w_b)
    rowsv = (rows_a, rows_b, rows_c)
    gsems = (gsem_a, gsem_b, gsem_c)
    wsems = (wsem_a, wsem_b)
    ssems = (ssem_a, ssem_b, ssem_c)
    _agg_section(c, s, xws2, src_sim, dst_sim, w_sim, out_sim,
                 S_SIM, acc_sh, idx_s, idx_d, wv, rowsv, gsems, wsems, ssems)
    _agg_section(c, s, xws2, src_dist, dst_dist, w_dist, out_dist,
                 S_SIM, acc_sh, idx_s, idx_d, wv, rowsv, gsems, wsems, ssems)


def _run_agg3(xws2, e_sim, e_dist):
    f = functools.partial(
        pl.kernel,
        mesh=_mesh(),
        out_type=[jax.ShapeDtypeStruct((NC, NPAD, 128), jnp.float32)] * 2,
        scratch_types=_agg_scratch(),
    )(_agg3_kernel)
    return f(xws2, e_sim[0], e_sim[1], e_sim[2],
             e_dist[0], e_dist[1], e_dist[2])


def _dinv(dp):
    deg = dp[0] + dp[1] + 1.0
    return jnp.where(deg > 0, lax.rsqrt(deg), 0.0)


def _t1_body(dps_ref, dpd_ref, dpc_ref, x_ref, adt_ref, atac_ref,
             w1_ref, w2_ref, wp_ref, wa_ref,
             xws_s_ref, xws_d_ref, xws_pa_ref, dvs_ref, dvd_ref, dvc_ref):
    dvs = _dinv(dps_ref[...])
    dvd = _dinv(dpd_ref[...])
    dvc = _dinv(dpc_ref[...])
    dvs_ref[...] = dvs
    dvd_ref[...] = dvd
    dvc_ref[...] = dvc
    x = x_ref[...]
    xws_s_ref[...] = jnp.dot(x, w1_ref[...], preferred_element_type=jnp.float32) * dvs
    xws_d_ref[...] = jnp.dot(x, w2_ref[...], preferred_element_type=jnp.float32) * dvd
    p = jnp.dot(adt_ref[...], wp_ref[...], preferred_element_type=jnp.float32) * dvc
    a = jnp.dot(atac_ref[...], wa_ref[...], preferred_element_type=jnp.float32) * dvc
    xws_pa_ref[...] = jnp.concatenate(
        [p, a, jnp.zeros((p.shape[0], 64), jnp.float32)], axis=1)


def _full(shape):
    nd = len(shape)
    return pl.BlockSpec(shape, lambda i: (0,) * nd)


def _rows(shape):
    nd = len(shape)
    return pl.BlockSpec(shape, lambda i: (i,) + (0,) * (nd - 1))


def _run_t1(dgp_sim, dgp_dist, dgp_com, x_rna, x_adt, x_atac, W1, W2, Wp, Wa):
    dps = dgp_sim[:, :N].reshape(NC, N, 1)
    dpd = dgp_dist[:, :N].reshape(NC, N, 1)
    dpc = dgp_com[:, :N].reshape(NC, N, 1)
    out_shape = [jax.ShapeDtypeStruct((N, 128), jnp.float32),
                 jax.ShapeDtypeStruct((N, 128), jnp.float32),
                 jax.ShapeDtypeStruct((N, 128), jnp.float32),
                 jax.ShapeDtypeStruct((N, 1), jnp.float32),
                 jax.ShapeDtypeStruct((N, 1), jnp.float32),
                 jax.ShapeDtypeStruct((N, 1), jnp.float32)]
    deg_spec = pl.BlockSpec((NC, BR, 1), lambda i: (0, i, 0))
    in_specs = [
        deg_spec, deg_spec, deg_spec,
        _rows((BR, 128)), _rows((BR, 32)), _rows((BR, 64)),
        _full((128, 128)), _full((128, 128)), _full((32, 32)), _full((64, 32)),
    ]
    out_specs = [_rows((BR, 128)), _rows((BR, 128)), _rows((BR, 128)),
                 _rows((BR, 1)), _rows((BR, 1)), _rows((BR, 1))]
    return pl.pallas_call(_t1_body, grid=(GRID,), in_specs=in_specs,
                          out_specs=out_specs, out_shape=out_shape)(
        dps, dpd, dpc, x_rna, x_adt, x_atac, W1, W2, Wp, Wa)


def _t2_body(accs_ref, accd_ref, xws_s_ref, xws_d_ref, dvs_ref, dvd_ref,
             b1_ref, b2_ref, ws_ref, wd_ref, xws2_ref):
    dvs = dvs_ref[...]
    dvd = dvd_ref[...]
    xs = jnp.maximum(dvs * (accs_ref[0] + accs_ref[1] + xws_s_ref[...]) + b1_ref[...], 0.0)
    xd = jnp.maximum(dvd * (accd_ref[0] + accd_ref[1] + xws_d_ref[...]) + b2_ref[...], 0.0)
    s2 = jnp.dot(xs, ws_ref[...], preferred_element_type=jnp.float32) * dvs
    d2 = jnp.dot(xd, wd_ref[...], preferred_element_type=jnp.float32) * dvd
    xws2_ref[...] = jnp.concatenate(
        [s2, d2, jnp.zeros((s2.shape[0], 64), jnp.float32)], axis=1)


def _run_t2(acc_sim, acc_dist, xws_sim, xws_dist, dvs, dvd, b1, b2, Ws, Wd):
    out_shape = [jax.ShapeDtypeStruct((N, 128), jnp.float32)]
    acc_spec = pl.BlockSpec((NC, BR, 128), lambda i: (0, i, 0))
    in_specs = [
        acc_spec, acc_spec,
        _rows((BR, 128)), _rows((BR, 128)), _rows((BR, 1)), _rows((BR, 1)),
        _full((1, 128)), _full((1, 128)), _full((128, 32)), _full((128, 32)),
    ]
    out_specs = [_rows((BR, 128))]
    return pl.pallas_call(_t2_body, grid=(GRID,), in_specs=in_specs,
                          out_specs=out_specs, out_shape=out_shape)(
        acc_sim[:, :N], acc_dist[:, :N], xws_sim, xws_dist, dvs, dvd,
        b1.reshape(1, 128), b2.reshape(1, 128), Ws, Wd)[0]


def _t3_body(acc2s_ref, acc2d_ref, accpa_ref, xws2_ref, xws_pa_ref,
             dvs_ref, dvd_ref, dvc_ref,
             bs_ref, bd_ref, bp_ref, ba_ref, bf_ref,
             wf1_ref, wf2_ref, wf3_ref, wf4_ref,
             xsim_ref, xdist_ref, fused_ref, pro_ref, atac_ref):
    dvs = dvs_ref[...]
    dvd = dvd_ref[...]
    dvc = dvc_ref[...]
    a2s = acc2s_ref[0, :, 0:32] + acc2s_ref[1, :, 0:32]
    a2d = acc2d_ref[0, :, 32:64] + acc2d_ref[1, :, 32:64]
    ap = accpa_ref[0, :, 0:32] + accpa_ref[1, :, 0:32]
    aa = accpa_ref[0, :, 32:64] + accpa_ref[1, :, 32:64]
    x_sim = dvs * (a2s + xws2_ref[:, 0:32]) + bs_ref[...]
    x_dist = dvd * (a2d + xws2_ref[:, 32:64]) + bd_ref[...]
    pro = dvc * (ap + xws_pa_ref[:, 0:32]) + bp_ref[...]
    atac = dvc * (aa + xws_pa_ref[:, 32:64]) + ba_ref[...]
    xsim_ref[...] = x_sim
    xdist_ref[...] = x_dist
    pro_ref[...] = pro
    atac_ref[...] = atac
    fused_ref[...] = (
        jnp.dot(x_sim, wf1_ref[...], preferred_element_type=jnp.float32)
        + jnp.dot(x_dist, wf2_ref[...], preferred_element_type=jnp.float32)
        + jnp.dot(pro, wf3_ref[...], preferred_element_type=jnp.float32)
        + jnp.dot(atac, wf4_ref[...], preferred_element_type=jnp.float32)
        + bf_ref[...])


def _run_t3(acc2_sim, acc2_dist, acc_pa, xws2, xws_pa, dvs, dvd, dvc,
            bs, bd, bp, ba, bf, Wf):
    out_shape = [jax.ShapeDtypeStruct((N, 32), jnp.float32)] * 5
    acc_spec = pl.BlockSpec((NC, BR, 128), lambda i: (0, i, 0))
    in_specs = [acc_spec] * 3 + [_rows((BR, 128))] * 2 + [_rows((BR, 1))] * 3 \
        + [_full((1, 32))] * 5 + [_full((32, 32))] * 4
    out_specs = [_rows((BR, 32))] * 5
    return pl.pallas_call(_t3_body, grid=(GRID,), in_specs=in_specs,
                          out_specs=out_specs, out_shape=out_shape)(
        acc2_sim[:, :N], acc2_dist[:, :N], acc_pa[:, :N], xws2, xws_pa,
        dvs, dvd, dvc,
        bs.reshape(1, 32), bd.reshape(1, 32), bp.reshape(1, 32),
        ba.reshape(1, 32), bf.reshape(1, 32),
        Wf[0:32], Wf[32:64], Wf[64:96], Wf[96:128])


def kernel(x_RNA, x_ADT, x_ATAC, sim_edge_index, sim_edge_weight,
           dist_edge_index, dist_edge_weight, common_edge_index,
           common_edge_weight, W_rna1, b_rna1, W_rna2, b_rna2, W_sim, b_sim,
           W_dist, b_dist, W_pro, b_pro, W_atac, b_atac, W_fuse, b_fuse):
    e_sim = _pad_edges(sim_edge_index, sim_edge_weight, S_SIM)
    e_dist = _pad_edges(dist_edge_index, dist_edge_weight, S_SIM)
    e_com = _pad_edges(common_edge_index, common_edge_weight, S_COM)

    dgp_sim, dgp_dist, dgp_com = _run_deg(
        e_sim[3], e_sim[4], e_dist[3], e_dist[4], e_com[3], e_com[4])

    xws_sim, xws_dist, xws_pa, dvs, dvd, dvc = _run_t1(
        dgp_sim, dgp_dist, dgp_com, x_RNA, x_ADT, x_ATAC,
        W_rna1, W_rna2, W_pro, W_atac)

    acc_sim, acc_dist, acc_pa = _run_agg2(
        xws_sim, xws_dist, xws_pa, e_sim, e_dist, e_com)

    xws2 = _run_t2(acc_sim, acc_dist, xws_sim, xws_dist, dvs, dvd,
                   b_rna1, b_rna2, W_sim, W_dist)

    acc2_sim, acc2_dist = _run_agg3(xws2, e_sim, e_dist)

    x_sim, x_dist, fused, pro, atac = _run_t3(
        acc2_sim, acc2_dist, acc_pa, xws2, xws_pa, dvs, dvd, dvc,
        b_sim, b_dist, b_pro, b_atac, b_fuse, W_fuse)
    return (x_sim, x_dist, fused, pro, atac)
